# Initial kernel scaffold; baseline (speedup 1.0000x reference)
import functools

import jax
import jax.numpy as jnp
from jax import lax
from jax.experimental import pallas as pl
from jax.experimental.pallas import tpu as pltpu
from jax.experimental.pallas import tpu_sc as plsc

N_NODES = 10000
D_FEAT = 128
N_EDGES = 320000

NC = 2
NS = 16
NW = NC * NS
K = 128
NCH = 79
E_PAD = NW * NCH * K
ROWS_PER_TILE = N_NODES // NS


@functools.partial(
    pl.kernel,
    out_type=jax.ShapeDtypeStruct((NC, N_NODES, D_FEAT), jnp.float32),
    mesh=plsc.VectorSubcoreMesh(core_axis_name="c", subcore_axis_name="s"),
    scratch_types=[
        pltpu.VMEM((NCH, K), jnp.int32),
        pltpu.VMEM((NCH, K), jnp.int32),
        pltpu.VMEM((NCH * K,), jnp.float32),
        pltpu.VMEM((K, D_FEAT), jnp.float32),
        pltpu.VMEM_SHARED((N_NODES, D_FEAT), jnp.float32),
        pltpu.SemaphoreType.DMA,
    ],
)
def _spmm_layer(h_hbm, src_hbm, dst_hbm, w_hbm, zeros_hbm, out_hbm,
                src_v, dst_v, w_v, rows_v, acc, sem):
    c = lax.axis_index("c")
    s = lax.axis_index("s")
    wid = s * NC + c

    pltpu.sync_copy(src_hbm.at[wid], src_v)
    pltpu.sync_copy(dst_hbm.at[wid], dst_v)
    pltpu.sync_copy(w_hbm.at[wid], w_v)

    pltpu.sync_copy(zeros_hbm.at[pl.ds(s * ROWS_PER_TILE, ROWS_PER_TILE)],
                    acc.at[pl.ds(s * ROWS_PER_TILE, ROWS_PER_TILE)])
    plsc.subcore_barrier()

    def chunk_body(j, carry):
        pltpu.async_copy(h_hbm.at[src_v.at[j]], rows_v, sem).wait()

        def grp_body(g, c2):
            base = j * K + g * 16
            for e in range(16):
                w_s = w_v[base + e]
                wb = jnp.full((16,), w_s, jnp.float32)
                row = g * 16 + e
                for r in range(D_FEAT // 16):
                    sl = pl.ds(r * 16, 16)
                    rows_v[row, sl] = rows_v[row, sl] * wb
            return c2

        lax.fori_loop(0, K // 16, grp_body, 0)

        pltpu.sync_copy(rows_v, acc.at[dst_v.at[j]], add=True)
        return carry

    lax.fori_loop(0, NCH, chunk_body, 0)
    plsc.subcore_barrier()

    pltpu.sync_copy(acc.at[pl.ds(s * ROWS_PER_TILE, ROWS_PER_TILE)],
                    out_hbm.at[c, pl.ds(s * ROWS_PER_TILE, ROWS_PER_TILE)])


def _add_body(p_ref, o_ref):
    o_ref[...] = p_ref[0] + p_ref[1]


_tc_add = pl.pallas_call(
    _add_body,
    grid=(10,),
    in_specs=[pl.BlockSpec((NC, N_NODES // 10, D_FEAT), lambda i: (0, i, 0))],
    out_specs=pl.BlockSpec((N_NODES // 10, D_FEAT), lambda i: (i, 0)),
    out_shape=jax.ShapeDtypeStruct((N_NODES, D_FEAT), jnp.float32),
)


def _fin_body(q_ref, w_ref, b_ref, o_ref):
    h = q_ref[0] + q_ref[1]
    o_ref[...] = jnp.dot(h, w_ref[...],
                         preferred_element_type=jnp.float32) + b_ref[...]


_tc_finish = pl.pallas_call(
    _fin_body,
    grid=(10,),
    in_specs=[
        pl.BlockSpec((NC, N_NODES // 10, D_FEAT), lambda i: (0, i, 0)),
        pl.BlockSpec((D_FEAT, D_FEAT), lambda i: (0, 0)),
        pl.BlockSpec((1, D_FEAT), lambda i: (0, 0)),
    ],
    out_specs=pl.BlockSpec((N_NODES // 10, D_FEAT), lambda i: (i, 0)),
    out_shape=jax.ShapeDtypeStruct((N_NODES, D_FEAT), jnp.float32),
)


def kernel(x, edge_index, edge_weight, W, b):
    src = edge_index[0].astype(jnp.int32)
    dst = edge_index[1].astype(jnp.int32)
    w = edge_weight.astype(jnp.float32)

    pad = E_PAD - N_EDGES
    src_p = jnp.concatenate([src, jnp.zeros((pad,), jnp.int32)]).reshape(NW, NCH, K)
    dst_p = jnp.concatenate([dst, jnp.zeros((pad,), jnp.int32)]).reshape(NW, NCH, K)
    w_p = jnp.concatenate([w, jnp.zeros((pad,), jnp.float32)]).reshape(NW, NCH * K)
    zeros = jnp.zeros((N_NODES, D_FEAT), jnp.float32)

    P = _spmm_layer(x, src_p, dst_p, w_p, zeros)
    h1 = _tc_add(P)
    Q = _spmm_layer(h1, src_p, dst_p, w_p, zeros)
    return _tc_finish(Q, W.astype(jnp.float32), b.reshape(1, D_FEAT))


# speedup vs baseline: 4.0283x; 4.0283x over previous
import functools

import jax
import jax.numpy as jnp
from jax import lax
from jax.experimental import pallas as pl
from jax.experimental.pallas import tpu as pltpu
from jax.experimental.pallas import tpu_sc as plsc

N_NODES = 10000
D_FEAT = 128
N_EDGES = 320000

NC = 2
NS = 16
NW = NC * NS
K = 128
NCH = 79
E_PAD = NW * NCH * K
ROWS_PER_TILE = 632


@functools.partial(
    pl.kernel,
    out_type=jax.ShapeDtypeStruct((NC, N_NODES, D_FEAT), jnp.float32),
    mesh=plsc.VectorSubcoreMesh(core_axis_name="c", subcore_axis_name="s"),
    scratch_types=[
        pltpu.VMEM((NCH, K), jnp.int32),
        pltpu.VMEM((NCH, K), jnp.int32),
        pltpu.VMEM((NCH * K,), jnp.float32),
        pltpu.VMEM((K, D_FEAT), jnp.float32),
        pltpu.VMEM_SHARED((N_NODES, D_FEAT), jnp.float32),
        pltpu.SemaphoreType.DMA,
    ],
)
def _spmm_layer(h_hbm, src_hbm, dst_hbm, w_hbm, zeros_hbm, out_hbm,
                src_v, dst_v, w_v, rows_v, acc, sem):
    c = lax.axis_index("c")
    s = lax.axis_index("s")
    wid = s * NC + c

    pltpu.sync_copy(src_hbm.at[wid], src_v)
    pltpu.sync_copy(dst_hbm.at[wid], dst_v)
    pltpu.sync_copy(w_hbm.at[wid], w_v)

    row_lo = jnp.minimum(s * ROWS_PER_TILE, N_NODES - ROWS_PER_TILE)

    pltpu.sync_copy(zeros_hbm.at[pl.ds(row_lo, ROWS_PER_TILE)],
                    acc.at[pl.ds(row_lo, ROWS_PER_TILE)])
    plsc.subcore_barrier()

    def chunk_body(j, carry):
        pltpu.async_copy(h_hbm.at[src_v.at[j]], rows_v, sem).wait()

        def grp_body(g, c2):
            base = j * K + g * 16
            w16 = w_v[pl.ds(base, 16)]
            for e in range(16):
                wb = jnp.full((16,), w16[e], jnp.float32)
                row = g * 16 + e
                for r in range(D_FEAT // 16):
                    sl = pl.ds(r * 16, 16)
                    rows_v[row, sl] = rows_v[row, sl] * wb
            return c2

        lax.fori_loop(0, K // 16, grp_body, 0)

        pltpu.sync_copy(rows_v, acc.at[dst_v.at[j]], add=True)
        return carry

    lax.fori_loop(0, NCH, chunk_body, 0)
    plsc.subcore_barrier()

    pltpu.sync_copy(acc.at[pl.ds(row_lo, ROWS_PER_TILE)],
                    out_hbm.at[c, pl.ds(row_lo, ROWS_PER_TILE)])


def _add_body(p_ref, o_ref):
    o_ref[...] = p_ref[0] + p_ref[1]


_tc_add = pl.pallas_call(
    _add_body,
    grid=(10,),
    in_specs=[pl.BlockSpec((NC, N_NODES // 10, D_FEAT), lambda i: (0, i, 0))],
    out_specs=pl.BlockSpec((N_NODES // 10, D_FEAT), lambda i: (i, 0)),
    out_shape=jax.ShapeDtypeStruct((N_NODES, D_FEAT), jnp.float32),
)


def _fin_body(q_ref, w_ref, b_ref, o_ref):
    h = q_ref[0] + q_ref[1]
    o_ref[...] = jnp.dot(h, w_ref[...],
                         preferred_element_type=jnp.float32) + b_ref[...]


_tc_finish = pl.pallas_call(
    _fin_body,
    grid=(10,),
    in_specs=[
        pl.BlockSpec((NC, N_NODES // 10, D_FEAT), lambda i: (0, i, 0)),
        pl.BlockSpec((D_FEAT, D_FEAT), lambda i: (0, 0)),
        pl.BlockSpec((1, D_FEAT), lambda i: (0, 0)),
    ],
    out_specs=pl.BlockSpec((N_NODES // 10, D_FEAT), lambda i: (i, 0)),
    out_shape=jax.ShapeDtypeStruct((N_NODES, D_FEAT), jnp.float32),
)


def kernel(x, edge_index, edge_weight, W, b):
    src = edge_index[0].astype(jnp.int32)
    dst = edge_index[1].astype(jnp.int32)
    w = edge_weight.astype(jnp.float32)

    pad = E_PAD - N_EDGES
    src_p = jnp.concatenate([src, jnp.zeros((pad,), jnp.int32)]).reshape(NW, NCH, K)
    dst_p = jnp.concatenate([dst, jnp.zeros((pad,), jnp.int32)]).reshape(NW, NCH, K)
    w_p = jnp.concatenate([w, jnp.zeros((pad,), jnp.float32)]).reshape(NW, NCH * K)
    zeros = jnp.zeros((N_NODES, D_FEAT), jnp.float32)

    P = _spmm_layer(x, src_p, dst_p, w_p, zeros)
    h1 = _tc_add(P)
    Q = _spmm_layer(h1, src_p, dst_p, w_p, zeros)
    return _tc_finish(Q, W.astype(jnp.float32), b.reshape(1, D_FEAT))
